# Initial kernel scaffold; baseline (speedup 1.0000x reference)
import jax
import jax.numpy as jnp
from jax.experimental import pallas as pl


def kernel(x, edge_index, W1, b1, W2, b2):
    raise NotImplementedError("write your pallas kernel here")


# speedup vs baseline: 34.2248x; 34.2248x over previous
import functools

import jax
import jax.numpy as jnp
from jax import lax
from jax.experimental import pallas as pl
from jax.experimental.pallas import tpu as pltpu
from jax.experimental.pallas import tpu_sc as plsc

N = 10000
E = 320000
D = 128
H = 16

NC = 2
NS = 16
NW = NC * NS

NPAD = 10112
ROWS_PER_SUB = NPAD // NS
EPAD = 327680
EDGES_PER_TILE = EPAD // NW
EROWS_PER_TILE = EDGES_PER_TILE // 128
CHUNK_ROWS = 16
N_CHUNKS = EROWS_PER_TILE // CHUNK_ROWS

_mesh = plsc.VectorSubcoreMesh(
    core_axis_name="c", subcore_axis_name="s", num_cores=NC, num_subcores=NS
)
_sc_params = pltpu.CompilerParams(use_tc_tiling_on_sc=False)


def _zero_shared_slice(zbuf, acc_sh, sid):
  def zbody(i, carry):
    zbuf[i, :] = jnp.zeros((16,), jnp.float32)
    return carry
  lax.fori_loop(0, ROWS_PER_SUB, zbody, 0)
  pltpu.sync_copy(zbuf, acc_sh.at[pl.ds(sid * ROWS_PER_SUB, ROWS_PER_SUB)])


@functools.partial(
    pl.kernel,
    out_type=jax.ShapeDtypeStruct((NC, NPAD, H), jnp.float32),
    mesh=_mesh,
    scratch_types=[
        pltpu.VMEM_SHARED((NPAD, H), jnp.float32),
        pltpu.VMEM((CHUNK_ROWS, 128), jnp.int32),
        pltpu.VMEM((128, H), jnp.float32),
        pltpu.VMEM((ROWS_PER_SUB, H), jnp.float32),
        pltpu.SemaphoreType.DMA,
    ],
    compiler_params=_sc_params,
)
def _deg_kernel(dst2d_hbm, out_hbm, acc_sh, dstv, ones_v, zbuf, ssem):
  cid = lax.axis_index("c")
  sid = lax.axis_index("s")
  tid = cid * NS + sid

  def obody(i, carry):
    ones_v[i, :] = jnp.ones((16,), jnp.float32)
    return carry
  lax.fori_loop(0, 128, obody, 0)
  _zero_shared_slice(zbuf, acc_sh, sid)
  plsc.subcore_barrier()

  def chunk_body(i, carry):
    r0 = tid * EROWS_PER_TILE + i * CHUNK_ROWS
    pltpu.sync_copy(dst2d_hbm.at[pl.ds(r0, CHUNK_ROWS)], dstv)
    descs = [
        pltpu.async_copy(ones_v, acc_sh.at[dstv.at[j]], ssem, add=True)
        for j in range(CHUNK_ROWS)
    ]
    for d in descs:
      d.wait()
    return carry

  lax.fori_loop(0, N_CHUNKS, chunk_body, 0)
  plsc.subcore_barrier()
  pltpu.sync_copy(
      acc_sh.at[pl.ds(sid * ROWS_PER_SUB, ROWS_PER_SUB)],
      out_hbm.at[cid, pl.ds(sid * ROWS_PER_SUB, ROWS_PER_SUB)],
  )


@functools.partial(
    pl.kernel,
    out_type=jax.ShapeDtypeStruct((NC, NPAD, H), jnp.float32),
    mesh=_mesh,
    scratch_types=[
        pltpu.VMEM_SHARED((NPAD, H), jnp.float32),
        pltpu.VMEM((CHUNK_ROWS, 128), jnp.int32),
        pltpu.VMEM((CHUNK_ROWS, 128), jnp.int32),
        pltpu.VMEM((CHUNK_ROWS * 128, H), jnp.float32),
        pltpu.VMEM((ROWS_PER_SUB, H), jnp.float32),
        pltpu.SemaphoreType.DMA,
        pltpu.SemaphoreType.DMA,
    ],
    compiler_params=_sc_params,
)
def _prop_kernel(hp_hbm, src2d_hbm, dst2d_hbm, out_hbm,
                 acc_sh, srcv, dstv, rows, zbuf, gsem, ssem):
  cid = lax.axis_index("c")
  sid = lax.axis_index("s")
  tid = cid * NS + sid

  _zero_shared_slice(zbuf, acc_sh, sid)
  plsc.subcore_barrier()

  def chunk_body(i, carry):
    r0 = tid * EROWS_PER_TILE + i * CHUNK_ROWS
    pltpu.sync_copy(src2d_hbm.at[pl.ds(r0, CHUNK_ROWS)], srcv)
    pltpu.sync_copy(dst2d_hbm.at[pl.ds(r0, CHUNK_ROWS)], dstv)
    gd = [
        pltpu.async_copy(
            hp_hbm.at[srcv.at[j]], rows.at[pl.ds(j * 128, 128)], gsem)
        for j in range(CHUNK_ROWS)
    ]
    for d in gd:
      d.wait()
    sd = [
        pltpu.async_copy(
            rows.at[pl.ds(j * 128, 128)], acc_sh.at[dstv.at[j]], ssem,
            add=True)
        for j in range(CHUNK_ROWS)
    ]
    for d in sd:
      d.wait()
    return carry

  lax.fori_loop(0, N_CHUNKS, chunk_body, 0)
  plsc.subcore_barrier()
  pltpu.sync_copy(
      acc_sh.at[pl.ds(sid * ROWS_PER_SUB, ROWS_PER_SUB)],
      out_hbm.at[cid, pl.ds(sid * ROWS_PER_SUB, ROWS_PER_SUB)],
  )


def _tc1_body(x_ref, w_ref, deg_ref, hp_ref):
  deg = deg_ref[0] + deg_ref[1] + 1.0
  dis = 1.0 / jnp.sqrt(deg)
  hp_ref[...] = dis * jnp.dot(
      x_ref[...], w_ref[...], preferred_element_type=jnp.float32)


def _tc2_body(acc_ref, hp_ref, deg_ref, w_ref, b_ref, out_ref):
  deg = deg_ref[0] + deg_ref[1] + 1.0
  dis = 1.0 / jnp.sqrt(deg)
  t = jnp.tanh(dis * (acc_ref[0] + acc_ref[1] + hp_ref[...]) + b_ref[...])
  out_ref[...] = dis * jnp.dot(t, w_ref[...],
                               preferred_element_type=jnp.float32)


def _tc3_body(acc_ref, hp_ref, deg_ref, b_ref, out_ref):
  deg = deg_ref[0] + deg_ref[1] + 1.0
  dis = 1.0 / jnp.sqrt(deg)
  out_ref[...] = jnp.tanh(
      dis * (acc_ref[0] + acc_ref[1] + hp_ref[...]) + b_ref[...])


def kernel(x, edge_index, W1, b1, W2, b2):
  f32 = jnp.float32
  src = edge_index[0].astype(jnp.int32)
  dst = edge_index[1].astype(jnp.int32)
  src_p = jnp.concatenate(
      [src, jnp.full((EPAD - E,), N, jnp.int32)]).reshape(EPAD // 128, 128)
  dst_p = jnp.concatenate(
      [dst, jnp.full((EPAD - E,), NPAD - 1, jnp.int32)]).reshape(
          EPAD // 128, 128)
  x_p = jnp.concatenate([x, jnp.zeros((NPAD - N, D), f32)], axis=0)

  deg = _deg_kernel(dst_p)

  hp1 = pl.pallas_call(
      _tc1_body,
      out_shape=jax.ShapeDtypeStruct((NPAD, H), f32),
  )(x_p, W1, deg)

  acc1 = _prop_kernel(hp1, src_p, dst_p)

  hp2 = pl.pallas_call(
      _tc2_body,
      out_shape=jax.ShapeDtypeStruct((NPAD, H), f32),
  )(acc1, hp1, deg, W2, b1.reshape(1, H))

  acc2 = _prop_kernel(hp2, src_p, dst_p)

  out = pl.pallas_call(
      _tc3_body,
      out_shape=jax.ShapeDtypeStruct((NPAD, H), f32),
  )(acc2, hp2, deg, b2.reshape(1, H))

  return out[:N]


# speedup vs baseline: 34.8129x; 1.0172x over previous
import functools

import jax
import jax.numpy as jnp
from jax import lax
from jax.experimental import pallas as pl
from jax.experimental.pallas import tpu as pltpu
from jax.experimental.pallas import tpu_sc as plsc

N = 10000
E = 320000
D = 128
H = 16

NC = 2
NS = 16
NW = NC * NS

NPAD = 10112
ROWS_PER_SUB = NPAD // NS
EPAD = 327680
EDGES_PER_TILE = EPAD // NW
EROWS_PER_TILE = EDGES_PER_TILE // 128
CHUNK_ROWS = 8
N_CHUNKS = EROWS_PER_TILE // CHUNK_ROWS
CHUNK_E = CHUNK_ROWS * 128

_mesh = plsc.VectorSubcoreMesh(
    core_axis_name="c", subcore_axis_name="s", num_cores=NC, num_subcores=NS
)
_sc_params = pltpu.CompilerParams(use_tc_tiling_on_sc=False)


def _zero_shared_slice(zbuf, acc_sh, sid):
  def zbody(i, carry):
    zbuf[i, :] = jnp.zeros((16,), jnp.float32)
    return carry
  lax.fori_loop(0, ROWS_PER_SUB, zbody, 0)
  pltpu.sync_copy(zbuf, acc_sh.at[pl.ds(sid * ROWS_PER_SUB, ROWS_PER_SUB)])


@functools.partial(
    pl.kernel,
    out_type=jax.ShapeDtypeStruct((NC, NPAD, H), jnp.float32),
    mesh=_mesh,
    scratch_types=[
        pltpu.VMEM_SHARED((NPAD, H), jnp.float32),
        pltpu.VMEM((CHUNK_ROWS, 128), jnp.int32),
        pltpu.VMEM((CHUNK_ROWS, 128), jnp.int32),
        pltpu.VMEM((128, H), jnp.float32),
        pltpu.VMEM((ROWS_PER_SUB, H), jnp.float32),
        pltpu.SemaphoreType.DMA,
        pltpu.SemaphoreType.DMA,
    ],
    compiler_params=_sc_params,
)
def _deg_kernel(dst2d_hbm, out_hbm, acc_sh, dstv0, dstv1, ones_v, zbuf,
                ssem0, ssem1):
  cid = lax.axis_index("c")
  sid = lax.axis_index("s")
  tid = cid * NS + sid
  base = tid * EROWS_PER_TILE
  dstv = (dstv0, dstv1)
  ssem = (ssem0, ssem1)

  def obody(i, carry):
    ones_v[i, :] = jnp.ones((16,), jnp.float32)
    return carry
  lax.fori_loop(0, 128, obody, 0)
  _zero_shared_slice(zbuf, acc_sh, sid)
  plsc.subcore_barrier()

  pltpu.sync_copy(dst2d_hbm.at[pl.ds(base, CHUNK_ROWS)], dstv[0])
  pending = [None, None]
  for c in range(N_CHUNKS):
    b = c % 2
    pending[b] = [
        pltpu.async_copy(ones_v, acc_sh.at[dstv[b].at[j]], ssem[b], add=True)
        for j in range(CHUNK_ROWS)
    ]
    if c + 1 < N_CHUNKS:
      if pending[1 - b] is not None:
        for d in pending[1 - b]:
          d.wait()
      pltpu.sync_copy(
          dst2d_hbm.at[pl.ds(base + (c + 1) * CHUNK_ROWS, CHUNK_ROWS)],
          dstv[1 - b])
  for p in pending:
    for d in p:
      d.wait()

  plsc.subcore_barrier()
  pltpu.sync_copy(
      acc_sh.at[pl.ds(sid * ROWS_PER_SUB, ROWS_PER_SUB)],
      out_hbm.at[cid, pl.ds(sid * ROWS_PER_SUB, ROWS_PER_SUB)],
  )


@functools.partial(
    pl.kernel,
    out_type=jax.ShapeDtypeStruct((NC, NPAD, H), jnp.float32),
    mesh=_mesh,
    scratch_types=[
        pltpu.VMEM_SHARED((NPAD, H), jnp.float32),
        pltpu.VMEM((CHUNK_ROWS, 128), jnp.int32),
        pltpu.VMEM((CHUNK_ROWS, 128), jnp.int32),
        pltpu.VMEM((CHUNK_ROWS, 128), jnp.int32),
        pltpu.VMEM((CHUNK_ROWS, 128), jnp.int32),
        pltpu.VMEM((CHUNK_E, H), jnp.float32),
        pltpu.VMEM((CHUNK_E, H), jnp.float32),
        pltpu.VMEM((ROWS_PER_SUB, H), jnp.float32),
        pltpu.SemaphoreType.DMA,
        pltpu.SemaphoreType.DMA,
        pltpu.SemaphoreType.DMA,
        pltpu.SemaphoreType.DMA,
    ],
    compiler_params=_sc_params,
)
def _prop_kernel(hp_hbm, src2d_hbm, dst2d_hbm, out_hbm,
                 acc_sh, srcv0, srcv1, dstv0, dstv1, rows0, rows1, zbuf,
                 gsem0, gsem1, ssem0, ssem1):
  cid = lax.axis_index("c")
  sid = lax.axis_index("s")
  tid = cid * NS + sid
  base = tid * EROWS_PER_TILE
  srcv = (srcv0, srcv1)
  dstv = (dstv0, dstv1)
  rows = (rows0, rows1)
  gsem = (gsem0, gsem1)
  ssem = (ssem0, ssem1)

  _zero_shared_slice(zbuf, acc_sh, sid)
  plsc.subcore_barrier()

  def load_idx(c, b):
    pltpu.sync_copy(
        src2d_hbm.at[pl.ds(base + c * CHUNK_ROWS, CHUNK_ROWS)], srcv[b])
    pltpu.sync_copy(
        dst2d_hbm.at[pl.ds(base + c * CHUNK_ROWS, CHUNK_ROWS)], dstv[b])

  def fire_gathers(b):
    return [
        pltpu.async_copy(
            hp_hbm.at[srcv[b].at[j]], rows[b].at[pl.ds(j * 128, 128)],
            gsem[b])
        for j in range(CHUNK_ROWS)
    ]

  def fire_scatters(b):
    return [
        pltpu.async_copy(
            rows[b].at[pl.ds(j * 128, 128)], acc_sh.at[dstv[b].at[j]],
            ssem[b], add=True)
        for j in range(CHUNK_ROWS)
    ]

  load_idx(0, 0)
  gpend = [None, None]
  spend = [None, None]
  gpend[0] = fire_gathers(0)
  for c in range(N_CHUNKS):
    b = c % 2
    for d in gpend[b]:
      d.wait()
    spend[b] = fire_scatters(b)
    if c + 1 < N_CHUNKS:
      if spend[1 - b] is not None:
        for d in spend[1 - b]:
          d.wait()
      load_idx(c + 1, 1 - b)
      gpend[1 - b] = fire_gathers(1 - b)
  for d in spend[0]:
    d.wait()
  for d in spend[1]:
    d.wait()

  plsc.subcore_barrier()
  pltpu.sync_copy(
      acc_sh.at[pl.ds(sid * ROWS_PER_SUB, ROWS_PER_SUB)],
      out_hbm.at[cid, pl.ds(sid * ROWS_PER_SUB, ROWS_PER_SUB)],
  )


def _tc1_body(x_ref, w_ref, deg_ref, hp_ref):
  deg = deg_ref[0] + deg_ref[1] + 1.0
  dis = 1.0 / jnp.sqrt(deg)
  hp_ref[...] = dis * jnp.dot(
      x_ref[...], w_ref[...], preferred_element_type=jnp.float32)


def _tc2_body(acc_ref, hp_ref, deg_ref, w_ref, b_ref, out_ref):
  deg = deg_ref[0] + deg_ref[1] + 1.0
  dis = 1.0 / jnp.sqrt(deg)
  t = jnp.tanh(dis * (acc_ref[0] + acc_ref[1] + hp_ref[...]) + b_ref[...])
  out_ref[...] = dis * jnp.dot(t, w_ref[...],
                               preferred_element_type=jnp.float32)


def _tc3_body(acc_ref, hp_ref, deg_ref, b_ref, out_ref):
  deg = deg_ref[0] + deg_ref[1] + 1.0
  dis = 1.0 / jnp.sqrt(deg)
  out_ref[...] = jnp.tanh(
      dis * (acc_ref[0] + acc_ref[1] + hp_ref[...]) + b_ref[...])


def kernel(x, edge_index, W1, b1, W2, b2):
  f32 = jnp.float32
  src = edge_index[0].astype(jnp.int32)
  dst = edge_index[1].astype(jnp.int32)
  src_p = jnp.concatenate(
      [src, jnp.full((EPAD - E,), N, jnp.int32)]).reshape(EPAD // 128, 128)
  dst_p = jnp.concatenate(
      [dst, jnp.full((EPAD - E,), NPAD - 1, jnp.int32)]).reshape(
          EPAD // 128, 128)
  x_p = jnp.concatenate([x, jnp.zeros((NPAD - N, D), f32)], axis=0)

  deg = _deg_kernel(dst_p)

  hp1 = pl.pallas_call(
      _tc1_body,
      out_shape=jax.ShapeDtypeStruct((NPAD, H), f32),
  )(x_p, W1, deg)

  acc1 = _prop_kernel(hp1, src_p, dst_p)

  hp2 = pl.pallas_call(
      _tc2_body,
      out_shape=jax.ShapeDtypeStruct((NPAD, H), f32),
  )(acc1, hp1, deg, W2, b1.reshape(1, H))

  acc2 = _prop_kernel(hp2, src_p, dst_p)

  out = pl.pallas_call(
      _tc3_body,
      out_shape=jax.ShapeDtypeStruct((NPAD, H), f32),
  )(acc2, hp2, deg, b2.reshape(1, H))

  return out[:N]


# speedup vs baseline: 40.2398x; 1.1559x over previous
import functools

import jax
import jax.numpy as jnp
from jax import lax
from jax.experimental import pallas as pl
from jax.experimental.pallas import tpu as pltpu
from jax.experimental.pallas import tpu_sc as plsc

N = 10000
E = 320000
D = 128
H = 16

NC = 2
NS = 16
NW = NC * NS

NPAD = 10112
ROWS_PER_SUB = NPAD // NS
EPAD = 327680
EDGES_PER_TILE = EPAD // NW
EROWS_PER_TILE = EDGES_PER_TILE // 128
CHUNK_ROWS = 8
N_CHUNKS = EROWS_PER_TILE // CHUNK_ROWS
CHUNK_E = CHUNK_ROWS * 128

_mesh = plsc.VectorSubcoreMesh(
    core_axis_name="c", subcore_axis_name="s", num_cores=NC, num_subcores=NS
)
_sc_params = pltpu.CompilerParams(use_tc_tiling_on_sc=False)


def _zero_shared_slice(zbuf, acc_sh, sid):
  def zbody(i, carry):
    zbuf[i, :] = jnp.zeros((16,), jnp.float32)
    return carry
  lax.fori_loop(0, ROWS_PER_SUB, zbody, 0)
  pltpu.sync_copy(zbuf, acc_sh.at[pl.ds(sid * ROWS_PER_SUB, ROWS_PER_SUB)])


@functools.partial(
    pl.kernel,
    out_type=jax.ShapeDtypeStruct((NC, NPAD, H), jnp.float32),
    mesh=_mesh,
    scratch_types=[
        pltpu.VMEM_SHARED((NPAD, H), jnp.float32),
        pltpu.VMEM((CHUNK_ROWS, 128), jnp.int32),
        pltpu.VMEM((CHUNK_ROWS, 128), jnp.int32),
        pltpu.VMEM((128, H), jnp.float32),
        pltpu.VMEM((ROWS_PER_SUB, H), jnp.float32),
        pltpu.SemaphoreType.DMA,
        pltpu.SemaphoreType.DMA,
    ],
    compiler_params=_sc_params,
)
def _deg_kernel(dst2d_hbm, out_hbm, acc_sh, dstv0, dstv1, ones_v, zbuf,
                ssem0, ssem1):
  cid = lax.axis_index("c")
  sid = lax.axis_index("s")
  tid = cid * NS + sid
  base = tid * EROWS_PER_TILE
  dstv = (dstv0, dstv1)
  ssem = (ssem0, ssem1)

  def obody(i, carry):
    ones_v[i, :] = jnp.ones((16,), jnp.float32)
    return carry
  lax.fori_loop(0, 128, obody, 0)
  _zero_shared_slice(zbuf, acc_sh, sid)
  plsc.subcore_barrier()

  pltpu.sync_copy(dst2d_hbm.at[pl.ds(base, CHUNK_ROWS)], dstv[0])
  pending = [None, None]
  for c in range(N_CHUNKS):
    b = c % 2
    pending[b] = [
        pltpu.async_copy(ones_v, acc_sh.at[dstv[b].at[j]], ssem[b], add=True)
        for j in range(CHUNK_ROWS)
    ]
    if c + 1 < N_CHUNKS:
      if pending[1 - b] is not None:
        for d in pending[1 - b]:
          d.wait()
      pltpu.sync_copy(
          dst2d_hbm.at[pl.ds(base + (c + 1) * CHUNK_ROWS, CHUNK_ROWS)],
          dstv[1 - b])
  for p in pending:
    for d in p:
      d.wait()

  plsc.subcore_barrier()
  pltpu.sync_copy(
      acc_sh.at[pl.ds(sid * ROWS_PER_SUB, ROWS_PER_SUB)],
      out_hbm.at[cid, pl.ds(sid * ROWS_PER_SUB, ROWS_PER_SUB)],
  )


@functools.partial(
    pl.kernel,
    out_type=jax.ShapeDtypeStruct((NC, NPAD, H), jnp.float32),
    mesh=_mesh,
    scratch_types=[
        pltpu.VMEM_SHARED((NPAD, H), jnp.float32),
        pltpu.VMEM((CHUNK_ROWS, 128), jnp.int32),
        pltpu.VMEM((CHUNK_ROWS, 128), jnp.int32),
        pltpu.VMEM((CHUNK_ROWS, 128), jnp.int32),
        pltpu.VMEM((CHUNK_ROWS, 128), jnp.int32),
        pltpu.VMEM((CHUNK_E, H), jnp.float32),
        pltpu.VMEM((CHUNK_E, H), jnp.float32),
        pltpu.VMEM((ROWS_PER_SUB, H), jnp.float32),
        pltpu.SemaphoreType.DMA,
        pltpu.SemaphoreType.DMA,
        pltpu.SemaphoreType.DMA,
        pltpu.SemaphoreType.DMA,
    ],
    compiler_params=_sc_params,
)
def _prop_kernel(hp_hbm, src2d_hbm, dst2d_hbm, out_hbm,
                 acc_sh, srcv0, srcv1, dstv0, dstv1, rows0, rows1, zbuf,
                 gsem0, gsem1, ssem0, ssem1):
  cid = lax.axis_index("c")
  sid = lax.axis_index("s")
  tid = cid * NS + sid
  base = tid * EROWS_PER_TILE
  srcv = (srcv0, srcv1)
  dstv = (dstv0, dstv1)
  rows = (rows0, rows1)
  gsem = (gsem0, gsem1)
  ssem = (ssem0, ssem1)

  _zero_shared_slice(zbuf, acc_sh, sid)
  plsc.subcore_barrier()

  def load_idx(c, b):
    pltpu.sync_copy(
        src2d_hbm.at[pl.ds(base + c * CHUNK_ROWS, CHUNK_ROWS)], srcv[b])
    pltpu.sync_copy(
        dst2d_hbm.at[pl.ds(base + c * CHUNK_ROWS, CHUNK_ROWS)], dstv[b])

  def fire_gathers(b):
    return [
        pltpu.async_copy(
            hp_hbm.at[srcv[b].at[j]], rows[b].at[pl.ds(j * 128, 128)],
            gsem[b])
        for j in range(CHUNK_ROWS)
    ]

  def fire_scatters(b):
    return [
        pltpu.async_copy(
            rows[b].at[pl.ds(j * 128, 128)], acc_sh.at[dstv[b].at[j]],
            ssem[b], add=True)
        for j in range(CHUNK_ROWS)
    ]

  load_idx(0, 0)
  gpend = [None, None]
  spend = [None, None]
  gpend[0] = fire_gathers(0)
  for c in range(N_CHUNKS):
    b = c % 2
    for d in gpend[b]:
      d.wait()
    spend[b] = fire_scatters(b)
    if c + 1 < N_CHUNKS:
      if spend[1 - b] is not None:
        for d in spend[1 - b]:
          d.wait()
      load_idx(c + 1, 1 - b)
      gpend[1 - b] = fire_gathers(1 - b)
  for d in spend[0]:
    d.wait()
  for d in spend[1]:
    d.wait()

  plsc.subcore_barrier()
  pltpu.sync_copy(
      acc_sh.at[pl.ds(sid * ROWS_PER_SUB, ROWS_PER_SUB)],
      out_hbm.at[cid, pl.ds(sid * ROWS_PER_SUB, ROWS_PER_SUB)],
  )


NP8 = NPAD // 8


def _tc1_body(x_ref, w_ref, deg_ref, hp_ref):
  deg = deg_ref[0] + deg_ref[1] + 1.0
  dis = 1.0 / jnp.sqrt(deg)
  hp_ref[...] = dis * jnp.dot(
      x_ref[...], w_ref[...], preferred_element_type=jnp.float32)


def _tc2_body(acc_ref, hp_ref, deg_ref, w_ref, b_ref, out_ref):
  deg = deg_ref[0] + deg_ref[1] + 1.0
  dis = 1.0 / jnp.sqrt(deg)
  t = jnp.tanh(dis * (acc_ref[0] + acc_ref[1] + hp_ref[...]) + b_ref[...])
  out_ref[...] = dis * jnp.dot(t, w_ref[...],
                               preferred_element_type=jnp.float32)


def _tc3_body(acc_ref, hp_ref, deg_ref, b_ref, out_ref):
  deg = deg_ref[0] + deg_ref[1] + 1.0
  dis = 1.0 / jnp.sqrt(deg)
  out_ref[...] = jnp.tanh(
      dis * (acc_ref[0] + acc_ref[1] + hp_ref[...]) + b_ref[...])


def kernel(x, edge_index, W1, b1, W2, b2):
  f32 = jnp.float32
  src = edge_index[0].astype(jnp.int32)
  dst = edge_index[1].astype(jnp.int32)
  src_p = jnp.concatenate(
      [src, jnp.full((EPAD - E,), N, jnp.int32)]).reshape(EPAD // 128, 128)
  dst_p = jnp.concatenate(
      [dst, jnp.full((EPAD - E,), NPAD - 1, jnp.int32)]).reshape(
          EPAD // 128, 128)
  x_p = jnp.concatenate([x, jnp.zeros((NPAD - N, D), f32)], axis=0)
  x8 = x_p.reshape(NP8, 8 * D)
  eye8 = jnp.eye(8, dtype=f32)
  W1b = jnp.kron(eye8, W1)
  W2b = jnp.kron(eye8, W2)
  b1t = jnp.tile(b1, 8).reshape(1, 128)
  b2t = jnp.tile(b2, 8).reshape(1, 128)

  deg = _deg_kernel(dst_p)
  deg8 = deg.reshape(NC, NP8, 128)

  hp1_8 = pl.pallas_call(
      _tc1_body,
      out_shape=jax.ShapeDtypeStruct((NP8, 128), f32),
  )(x8, W1b, deg8)

  acc1 = _prop_kernel(hp1_8.reshape(NPAD, H), src_p, dst_p)

  hp2_8 = pl.pallas_call(
      _tc2_body,
      out_shape=jax.ShapeDtypeStruct((NP8, 128), f32),
  )(acc1.reshape(NC, NP8, 128), hp1_8, deg8, W2b, b1t)

  acc2 = _prop_kernel(hp2_8.reshape(NPAD, H), src_p, dst_p)

  out8 = pl.pallas_call(
      _tc3_body,
      out_shape=jax.ShapeDtypeStruct((NP8, 128), f32),
  )(acc2.reshape(NC, NP8, 128), hp2_8, deg8, b2t)

  return out8.reshape(NPAD, H)[:N]


# speedup vs baseline: 70.4074x; 1.7497x over previous
import functools

import jax
import jax.numpy as jnp
from jax import lax
from jax.experimental import pallas as pl
from jax.experimental.pallas import tpu as pltpu
from jax.experimental.pallas import tpu_sc as plsc

N = 10000
E = 320000
D = 128
H = 16

NC = 2
NS = 16
NW = NC * NS

NPAD = 10112
ROWS_PER_SUB = NPAD // NS
EPAD = 327680
EDGES_PER_TILE = EPAD // NW
EROWS_PER_TILE = EDGES_PER_TILE // 128
CHUNK_ROWS = 8
N_CHUNKS = EROWS_PER_TILE // CHUNK_ROWS
CHUNK_E = CHUNK_ROWS * 128

_mesh = plsc.VectorSubcoreMesh(
    core_axis_name="c", subcore_axis_name="s", num_cores=NC, num_subcores=NS
)
_sc_params = pltpu.CompilerParams(use_tc_tiling_on_sc=False)


def _zero_shared_slice(zbuf, acc_sh, sid):
  def zbody(i, carry):
    zbuf[i, :] = jnp.zeros((16,), jnp.float32)
    return carry
  lax.fori_loop(0, ROWS_PER_SUB, zbody, 0)
  pltpu.sync_copy(zbuf, acc_sh.at[pl.ds(sid * ROWS_PER_SUB, ROWS_PER_SUB)])


@functools.partial(
    pl.kernel,
    out_type=jax.ShapeDtypeStruct((NC, NPAD, H), jnp.float32),
    mesh=_mesh,
    scratch_types=[
        pltpu.VMEM_SHARED((NPAD, H), jnp.float32),
        pltpu.VMEM((CHUNK_ROWS, 128), jnp.int32),
        pltpu.VMEM((CHUNK_ROWS, 128), jnp.int32),
        pltpu.VMEM((128, H), jnp.float32),
        pltpu.VMEM((ROWS_PER_SUB, H), jnp.float32),
        pltpu.SemaphoreType.DMA,
        pltpu.SemaphoreType.DMA,
    ],
    compiler_params=_sc_params,
)
def _deg_kernel(dst2d_hbm, out_hbm, acc_sh, dstv0, dstv1, ones_v, zbuf,
                ssem0, ssem1):
  cid = lax.axis_index("c")
  sid = lax.axis_index("s")
  tid = cid * NS + sid
  base = tid * EROWS_PER_TILE
  dstv = (dstv0, dstv1)
  ssem = (ssem0, ssem1)

  def obody(i, carry):
    ones_v[i, :] = jnp.ones((16,), jnp.float32)
    return carry
  lax.fori_loop(0, 128, obody, 0)
  _zero_shared_slice(zbuf, acc_sh, sid)
  plsc.subcore_barrier()

  pltpu.sync_copy(dst2d_hbm.at[pl.ds(base, CHUNK_ROWS)], dstv[0])
  pending = [None, None]
  for c in range(N_CHUNKS):
    b = c % 2
    pending[b] = [
        pltpu.async_copy(ones_v, acc_sh.at[dstv[b].at[j]], ssem[b], add=True)
        for j in range(CHUNK_ROWS)
    ]
    if c + 1 < N_CHUNKS:
      if pending[1 - b] is not None:
        for d in pending[1 - b]:
          d.wait()
      pltpu.sync_copy(
          dst2d_hbm.at[pl.ds(base + (c + 1) * CHUNK_ROWS, CHUNK_ROWS)],
          dstv[1 - b])
  for p in pending:
    for d in p:
      d.wait()

  plsc.subcore_barrier()
  pltpu.sync_copy(
      acc_sh.at[pl.ds(sid * ROWS_PER_SUB, ROWS_PER_SUB)],
      out_hbm.at[cid, pl.ds(sid * ROWS_PER_SUB, ROWS_PER_SUB)],
  )


@functools.partial(
    pl.kernel,
    out_type=jax.ShapeDtypeStruct((NC, NPAD, H), jnp.float32),
    mesh=_mesh,
    scratch_types=[
        pltpu.VMEM_SHARED((NPAD, H), jnp.float32),
        pltpu.VMEM_SHARED((NPAD, H), jnp.float32),
        pltpu.VMEM((CHUNK_ROWS, 128), jnp.int32),
        pltpu.VMEM((CHUNK_ROWS, 128), jnp.int32),
        pltpu.VMEM((CHUNK_ROWS, 128), jnp.int32),
        pltpu.VMEM((CHUNK_ROWS, 128), jnp.int32),
        pltpu.VMEM((CHUNK_E, H), jnp.float32),
        pltpu.VMEM((CHUNK_E, H), jnp.float32),
        pltpu.VMEM((ROWS_PER_SUB, H), jnp.float32),
        pltpu.SemaphoreType.DMA,
        pltpu.SemaphoreType.DMA,
        pltpu.SemaphoreType.DMA,
        pltpu.SemaphoreType.DMA,
    ],
    compiler_params=_sc_params,
)
def _prop_kernel(hp_hbm, src2d_hbm, dst2d_hbm, out_hbm,
                 acc_sh, hp_sh, srcv0, srcv1, dstv0, dstv1, rows0, rows1, zbuf,
                 gsem0, gsem1, ssem0, ssem1):
  cid = lax.axis_index("c")
  sid = lax.axis_index("s")
  tid = cid * NS + sid
  base = tid * EROWS_PER_TILE
  srcv = (srcv0, srcv1)
  dstv = (dstv0, dstv1)
  rows = (rows0, rows1)
  gsem = (gsem0, gsem1)
  ssem = (ssem0, ssem1)

  _zero_shared_slice(zbuf, acc_sh, sid)
  pltpu.sync_copy(
      hp_hbm.at[pl.ds(sid * ROWS_PER_SUB, ROWS_PER_SUB)],
      hp_sh.at[pl.ds(sid * ROWS_PER_SUB, ROWS_PER_SUB)])
  plsc.subcore_barrier()

  def load_idx(c, b):
    pltpu.sync_copy(
        src2d_hbm.at[pl.ds(base + c * CHUNK_ROWS, CHUNK_ROWS)], srcv[b])
    pltpu.sync_copy(
        dst2d_hbm.at[pl.ds(base + c * CHUNK_ROWS, CHUNK_ROWS)], dstv[b])

  def fire_gathers(b):
    return [
        pltpu.async_copy(
            hp_sh.at[srcv[b].at[j]], rows[b].at[pl.ds(j * 128, 128)],
            gsem[b])
        for j in range(CHUNK_ROWS)
    ]

  def fire_scatters(b):
    return [
        pltpu.async_copy(
            rows[b].at[pl.ds(j * 128, 128)], acc_sh.at[dstv[b].at[j]],
            ssem[b], add=True)
        for j in range(CHUNK_ROWS)
    ]

  load_idx(0, 0)
  gpend = [None, None]
  spend = [None, None]
  gpend[0] = fire_gathers(0)
  for c in range(N_CHUNKS):
    b = c % 2
    for d in gpend[b]:
      d.wait()
    spend[b] = fire_scatters(b)
    if c + 1 < N_CHUNKS:
      if spend[1 - b] is not None:
        for d in spend[1 - b]:
          d.wait()
      load_idx(c + 1, 1 - b)
      gpend[1 - b] = fire_gathers(1 - b)
  for d in spend[0]:
    d.wait()
  for d in spend[1]:
    d.wait()

  plsc.subcore_barrier()
  pltpu.sync_copy(
      acc_sh.at[pl.ds(sid * ROWS_PER_SUB, ROWS_PER_SUB)],
      out_hbm.at[cid, pl.ds(sid * ROWS_PER_SUB, ROWS_PER_SUB)],
  )


NP8 = NPAD // 8


def _tc1_body(x_ref, w_ref, deg_ref, hp_ref):
  deg = deg_ref[0] + deg_ref[1] + 1.0
  dis = 1.0 / jnp.sqrt(deg)
  hp_ref[...] = dis * jnp.dot(
      x_ref[...], w_ref[...], preferred_element_type=jnp.float32)


def _tc2_body(acc_ref, hp_ref, deg_ref, w_ref, b_ref, out_ref):
  deg = deg_ref[0] + deg_ref[1] + 1.0
  dis = 1.0 / jnp.sqrt(deg)
  t = jnp.tanh(dis * (acc_ref[0] + acc_ref[1] + hp_ref[...]) + b_ref[...])
  out_ref[...] = dis * jnp.dot(t, w_ref[...],
                               preferred_element_type=jnp.float32)


def _tc3_body(acc_ref, hp_ref, deg_ref, b_ref, out_ref):
  deg = deg_ref[0] + deg_ref[1] + 1.0
  dis = 1.0 / jnp.sqrt(deg)
  out_ref[...] = jnp.tanh(
      dis * (acc_ref[0] + acc_ref[1] + hp_ref[...]) + b_ref[...])


def kernel(x, edge_index, W1, b1, W2, b2):
  f32 = jnp.float32
  src = edge_index[0].astype(jnp.int32)
  dst = edge_index[1].astype(jnp.int32)
  src_p = jnp.concatenate(
      [src, jnp.full((EPAD - E,), N, jnp.int32)]).reshape(EPAD // 128, 128)
  dst_p = jnp.concatenate(
      [dst, jnp.full((EPAD - E,), NPAD - 1, jnp.int32)]).reshape(
          EPAD // 128, 128)
  x_p = jnp.concatenate([x, jnp.zeros((NPAD - N, D), f32)], axis=0)
  x8 = x_p.reshape(NP8, 8 * D)
  eye8 = jnp.eye(8, dtype=f32)
  W1b = jnp.kron(eye8, W1)
  W2b = jnp.kron(eye8, W2)
  b1t = jnp.tile(b1, 8).reshape(1, 128)
  b2t = jnp.tile(b2, 8).reshape(1, 128)

  deg = _deg_kernel(dst_p)
  deg8 = deg.reshape(NC, NP8, 128)

  hp1_8 = pl.pallas_call(
      _tc1_body,
      out_shape=jax.ShapeDtypeStruct((NP8, 128), f32),
  )(x8, W1b, deg8)

  acc1 = _prop_kernel(hp1_8.reshape(NPAD, H), src_p, dst_p)

  hp2_8 = pl.pallas_call(
      _tc2_body,
      out_shape=jax.ShapeDtypeStruct((NP8, 128), f32),
  )(acc1.reshape(NC, NP8, 128), hp1_8, deg8, W2b, b1t)

  acc2 = _prop_kernel(hp2_8.reshape(NPAD, H), src_p, dst_p)

  out8 = pl.pallas_call(
      _tc3_body,
      out_shape=jax.ShapeDtypeStruct((NP8, 128), f32),
  )(acc2.reshape(NC, NP8, 128), hp2_8, deg8, b2t)

  return out8.reshape(NPAD, H)[:N]


# speedup vs baseline: 83.7362x; 1.1893x over previous
import functools

import jax
import jax.numpy as jnp
from jax import lax
from jax.experimental import pallas as pl
from jax.experimental.pallas import tpu as pltpu
from jax.experimental.pallas import tpu_sc as plsc

N = 10000
E = 320000
D = 128
H = 16

NC = 2
NS = 16
NW = NC * NS

NPAD = 10112
ROWS_PER_SUB = NPAD // NS
MAIN_ROWS = 78
CHUNK_ROWS = 13
N_CHUNKS = MAIN_ROWS // CHUNK_ROWS
CHUNK_E = CHUNK_ROWS * 128
TAIL_E0 = MAIN_ROWS * NW * 128
TAIL_E = E - TAIL_E0
TAIL_OWNERS = (NS - 1, NW - 1)

_mesh = plsc.VectorSubcoreMesh(
    core_axis_name="c", subcore_axis_name="s", num_cores=NC, num_subcores=NS
)
_sc_params = pltpu.CompilerParams(use_tc_tiling_on_sc=False)


@functools.partial(
    pl.kernel,
    out_type=jax.ShapeDtypeStruct((NC, NPAD, H), jnp.float32),
    mesh=_mesh,
    scratch_types=[
        pltpu.VMEM_SHARED((NPAD, H), jnp.float32),
        pltpu.VMEM((CHUNK_E,), jnp.int32),
        pltpu.VMEM((CHUNK_E,), jnp.int32),
        pltpu.VMEM((TAIL_E,), jnp.int32),
        pltpu.VMEM((CHUNK_E, H), jnp.float32),
        pltpu.VMEM((TAIL_E, H), jnp.float32),
        pltpu.SemaphoreType.DMA,
        pltpu.SemaphoreType.DMA,
    ],
    compiler_params=_sc_params,
)
def _deg_kernel(ei_hbm, zeros_hbm, ones_hbm, out_hbm,
                acc_sh, dstv0, dstv1, dstvt, ones_v, mones, ssem0, ssem1):
  cid = lax.axis_index("c")
  sid = lax.axis_index("s")
  tid = cid * NS + sid
  base = tid * MAIN_ROWS * 128
  dstv = (dstv0, dstv1)
  ssem = (ssem0, ssem1)
  m0 = jnp.where(tid == TAIL_OWNERS[0], 1.0, 0.0).astype(jnp.float32)
  m1 = jnp.where(tid == TAIL_OWNERS[1], 1.0, 0.0).astype(jnp.float32)

  pltpu.sync_copy(ones_hbm, ones_v)

  def mbody(i, carry):
    mones[i, :] = ones_v[i, :] * m0
    mones[256 + i, :] = ones_v[256 + i, :] * m1
    return carry
  lax.fori_loop(0, TAIL_E // 2, mbody, 0)

  pltpu.sync_copy(
      zeros_hbm, acc_sh.at[pl.ds(sid * ROWS_PER_SUB, ROWS_PER_SUB)])
  plsc.subcore_barrier()

  pltpu.sync_copy(ei_hbm.at[1, pl.ds(base, CHUNK_E)], dstv[0])
  pending = [None, None]
  for c in range(N_CHUNKS):
    b = c % 2
    pending[b] = pltpu.async_copy(
        ones_v, acc_sh.at[dstv[b]], ssem[b], add=True)
    if c + 1 < N_CHUNKS:
      if pending[1 - b] is not None:
        pending[1 - b].wait()
      pltpu.sync_copy(
          ei_hbm.at[1, pl.ds(base + (c + 1) * CHUNK_E, CHUNK_E)],
          dstv[1 - b])
  pltpu.sync_copy(ei_hbm.at[1, pl.ds(TAIL_E0, TAIL_E)], dstvt)
  tail = pltpu.async_copy(mones, acc_sh.at[dstvt], ssem[0], add=True)
  pending[0].wait()
  pending[1].wait()
  tail.wait()

  plsc.subcore_barrier()
  pltpu.sync_copy(
      acc_sh.at[pl.ds(sid * ROWS_PER_SUB, ROWS_PER_SUB)],
      out_hbm.at[cid, pl.ds(sid * ROWS_PER_SUB, ROWS_PER_SUB)],
  )


@functools.partial(
    pl.kernel,
    out_type=jax.ShapeDtypeStruct((NC, NPAD, H), jnp.float32),
    mesh=_mesh,
    scratch_types=[
        pltpu.VMEM_SHARED((NPAD, H), jnp.float32),
        pltpu.VMEM_SHARED((NPAD, H), jnp.float32),
        pltpu.VMEM((CHUNK_E,), jnp.int32),
        pltpu.VMEM((CHUNK_E,), jnp.int32),
        pltpu.VMEM((CHUNK_E,), jnp.int32),
        pltpu.VMEM((CHUNK_E,), jnp.int32),
        pltpu.VMEM((TAIL_E,), jnp.int32),
        pltpu.VMEM((TAIL_E,), jnp.int32),
        pltpu.VMEM((CHUNK_E, H), jnp.float32),
        pltpu.VMEM((CHUNK_E, H), jnp.float32),
        pltpu.SemaphoreType.DMA,
        pltpu.SemaphoreType.DMA,
        pltpu.SemaphoreType.DMA,
        pltpu.SemaphoreType.DMA,
    ],
    compiler_params=_sc_params,
)
def _prop_kernel(hp_hbm, ei_hbm, zeros_hbm, out_hbm,
                 acc_sh, hp_sh, srcv0, srcv1, dstv0, dstv1, srcvt, dstvt,
                 rows0, rows1, gsem0, gsem1, ssem0, ssem1):
  cid = lax.axis_index("c")
  sid = lax.axis_index("s")
  tid = cid * NS + sid
  base = tid * MAIN_ROWS * 128
  srcv = (srcv0, srcv1)
  dstv = (dstv0, dstv1)
  rows = (rows0, rows1)
  gsem = (gsem0, gsem1)
  ssem = (ssem0, ssem1)
  m0 = jnp.where(tid == TAIL_OWNERS[0], 1.0, 0.0).astype(jnp.float32)
  m1 = jnp.where(tid == TAIL_OWNERS[1], 1.0, 0.0).astype(jnp.float32)

  pltpu.sync_copy(
      zeros_hbm, acc_sh.at[pl.ds(sid * ROWS_PER_SUB, ROWS_PER_SUB)])
  pltpu.sync_copy(
      hp_hbm.at[pl.ds(sid * ROWS_PER_SUB, ROWS_PER_SUB)],
      hp_sh.at[pl.ds(sid * ROWS_PER_SUB, ROWS_PER_SUB)])
  plsc.subcore_barrier()

  def load_idx(c, b):
    off = base + c * CHUNK_E
    pltpu.sync_copy(ei_hbm.at[0, pl.ds(off, CHUNK_E)], srcv[b])
    pltpu.sync_copy(ei_hbm.at[1, pl.ds(off, CHUNK_E)], dstv[b])

  load_idx(0, 0)
  gpend = [None, None]
  spend = [None, None]
  gpend[0] = pltpu.async_copy(hp_sh.at[srcv[0]], rows[0], gsem[0])
  for c in range(N_CHUNKS):
    b = c % 2
    gpend[b].wait()
    spend[b] = pltpu.async_copy(
        rows[b], acc_sh.at[dstv[b]], ssem[b], add=True)
    if c + 1 < N_CHUNKS:
      if spend[1 - b] is not None:
        spend[1 - b].wait()
      load_idx(c + 1, 1 - b)
      gpend[1 - b] = pltpu.async_copy(
          hp_sh.at[srcv[1 - b]], rows[1 - b], gsem[1 - b])
  spend[0].wait()
  spend[1].wait()

  pltpu.sync_copy(ei_hbm.at[0, pl.ds(TAIL_E0, TAIL_E)], srcvt)
  pltpu.sync_copy(ei_hbm.at[1, pl.ds(TAIL_E0, TAIL_E)], dstvt)
  pltpu.async_copy(
      hp_sh.at[srcvt], rows0.at[pl.ds(0, TAIL_E)], gsem[0]).wait()

  def mbody(i, carry):
    rows0[i, :] = rows0[i, :] * m0
    rows0[256 + i, :] = rows0[256 + i, :] * m1
    return carry
  lax.fori_loop(0, TAIL_E // 2, mbody, 0)
  pltpu.async_copy(
      rows0.at[pl.ds(0, TAIL_E)], acc_sh.at[dstvt], ssem[0],
      add=True).wait()

  plsc.subcore_barrier()
  pltpu.sync_copy(
      acc_sh.at[pl.ds(sid * ROWS_PER_SUB, ROWS_PER_SUB)],
      out_hbm.at[cid, pl.ds(sid * ROWS_PER_SUB, ROWS_PER_SUB)],
  )


NP8 = NPAD // 8


def _tc1_body(x_ref, w_ref, deg_ref, hp_ref):
  deg = deg_ref[0] + deg_ref[1] + 1.0
  dis = 1.0 / jnp.sqrt(deg)
  hp_ref[...] = dis * jnp.dot(
      x_ref[...], w_ref[...], preferred_element_type=jnp.float32)


def _tc2_body(acc_ref, hp_ref, deg_ref, w_ref, b_ref, out_ref):
  deg = deg_ref[0] + deg_ref[1] + 1.0
  dis = 1.0 / jnp.sqrt(deg)
  t = jnp.tanh(dis * (acc_ref[0] + acc_ref[1] + hp_ref[...]) + b_ref[...])
  out_ref[...] = dis * jnp.dot(t, w_ref[...],
                               preferred_element_type=jnp.float32)


def _tc3_body(acc_ref, hp_ref, deg_ref, b_ref, out_ref):
  deg = deg_ref[0] + deg_ref[1] + 1.0
  dis = 1.0 / jnp.sqrt(deg)
  out_ref[...] = jnp.tanh(
      dis * (acc_ref[0] + acc_ref[1] + hp_ref[...]) + b_ref[...])


def kernel(x, edge_index, W1, b1, W2, b2):
  f32 = jnp.float32
  ei = edge_index.astype(jnp.int32)
  x_p = jnp.concatenate([x, jnp.zeros((NPAD - N, D), f32)], axis=0)
  zeros_hbm = jnp.zeros((ROWS_PER_SUB, H), f32)
  ones_hbm = jnp.ones((CHUNK_E, H), f32)
  x8 = x_p.reshape(NP8, 8 * D)
  eye8 = jnp.eye(8, dtype=f32)
  W1b = jnp.kron(eye8, W1)
  W2b = jnp.kron(eye8, W2)
  b1t = jnp.tile(b1, 8).reshape(1, 128)
  b2t = jnp.tile(b2, 8).reshape(1, 128)

  deg = _deg_kernel(ei, zeros_hbm, ones_hbm)
  deg8 = deg.reshape(NC, NP8, 128)

  hp1_8 = pl.pallas_call(
      _tc1_body,
      out_shape=jax.ShapeDtypeStruct((NP8, 128), f32),
  )(x8, W1b, deg8)

  acc1 = _prop_kernel(hp1_8.reshape(NPAD, H), ei, zeros_hbm)

  hp2_8 = pl.pallas_call(
      _tc2_body,
      out_shape=jax.ShapeDtypeStruct((NP8, 128), f32),
  )(acc1.reshape(NC, NP8, 128), hp1_8, deg8, W2b, b1t)

  acc2 = _prop_kernel(hp2_8.reshape(NPAD, H), ei, zeros_hbm)

  out8 = pl.pallas_call(
      _tc3_body,
      out_shape=jax.ShapeDtypeStruct((NP8, 128), f32),
  )(acc2.reshape(NC, NP8, 128), hp2_8, deg8, b2t)

  return out8.reshape(NPAD, H)[:N]


# speedup vs baseline: 84.9075x; 1.0140x over previous
import functools

import jax
import jax.numpy as jnp
from jax import lax
from jax.experimental import pallas as pl
from jax.experimental.pallas import tpu as pltpu
from jax.experimental.pallas import tpu_sc as plsc

N = 10000
E = 320000
D = 128
H = 16

NC = 2
NS = 16
NW = NC * NS

NPAD = 10112
ROWS_PER_SUB = NPAD // NS
MAIN_ROWS = 78
CHUNK_ROWS = 13
N_CHUNKS = MAIN_ROWS // CHUNK_ROWS
CHUNK_E = CHUNK_ROWS * 128
TAIL_E0 = MAIN_ROWS * NW * 128
TAIL_E = E - TAIL_E0
TAIL_OWNERS = (NS - 1, NW - 1)

_mesh = plsc.VectorSubcoreMesh(
    core_axis_name="c", subcore_axis_name="s", num_cores=NC, num_subcores=NS
)
_sc_params = pltpu.CompilerParams(use_tc_tiling_on_sc=False)


@functools.partial(
    pl.kernel,
    out_type=jax.ShapeDtypeStruct((NC, NPAD, H), jnp.float32),
    mesh=_mesh,
    scratch_types=[
        pltpu.VMEM_SHARED((NPAD, H), jnp.float32),
        pltpu.VMEM((CHUNK_E,), jnp.int32),
        pltpu.VMEM((CHUNK_E,), jnp.int32),
        pltpu.VMEM((TAIL_E,), jnp.int32),
        pltpu.VMEM((CHUNK_E, H), jnp.float32),
        pltpu.VMEM((TAIL_E, H), jnp.float32),
        pltpu.SemaphoreType.DMA,
        pltpu.SemaphoreType.DMA,
    ],
    compiler_params=_sc_params,
)
def _deg_kernel(ei_hbm, zeros_hbm, ones_hbm, out_hbm,
                acc_sh, dstv0, dstv1, dstvt, ones_v, mones, ssem0, ssem1):
  cid = lax.axis_index("c")
  sid = lax.axis_index("s")
  tid = cid * NS + sid
  base = tid * MAIN_ROWS * 128
  dstv = (dstv0, dstv1)
  ssem = (ssem0, ssem1)
  m0 = jnp.where(tid == TAIL_OWNERS[0], 1.0, 0.0).astype(jnp.float32)
  m1 = jnp.where(tid == TAIL_OWNERS[1], 1.0, 0.0).astype(jnp.float32)

  pltpu.sync_copy(ones_hbm, ones_v)

  def mbody(i, carry):
    mones[i, :] = ones_v[i, :] * m0
    mones[256 + i, :] = ones_v[256 + i, :] * m1
    return carry
  lax.fori_loop(0, TAIL_E // 2, mbody, 0)

  pltpu.sync_copy(
      zeros_hbm, acc_sh.at[pl.ds(sid * ROWS_PER_SUB, ROWS_PER_SUB)])
  plsc.subcore_barrier()

  pltpu.sync_copy(ei_hbm.at[1, pl.ds(base, CHUNK_E)], dstv[0])
  pending = [None, None]
  for c in range(N_CHUNKS):
    b = c % 2
    pending[b] = pltpu.async_copy(
        ones_v, acc_sh.at[dstv[b]], ssem[b], add=True)
    if c + 1 < N_CHUNKS:
      if pending[1 - b] is not None:
        pending[1 - b].wait()
      pltpu.sync_copy(
          ei_hbm.at[1, pl.ds(base + (c + 1) * CHUNK_E, CHUNK_E)],
          dstv[1 - b])
  pltpu.sync_copy(ei_hbm.at[1, pl.ds(TAIL_E0, TAIL_E)], dstvt)
  tail = pltpu.async_copy(mones, acc_sh.at[dstvt], ssem[0], add=True)
  pending[0].wait()
  pending[1].wait()
  tail.wait()

  plsc.subcore_barrier()
  pltpu.sync_copy(
      acc_sh.at[pl.ds(sid * ROWS_PER_SUB, ROWS_PER_SUB)],
      out_hbm.at[cid, pl.ds(sid * ROWS_PER_SUB, ROWS_PER_SUB)],
  )


@functools.partial(
    pl.kernel,
    out_type=jax.ShapeDtypeStruct((NC, NPAD, H), jnp.float32),
    mesh=_mesh,
    scratch_types=[
        pltpu.VMEM_SHARED((NPAD, H), jnp.float32),
        pltpu.VMEM_SHARED((NPAD, H), jnp.float32),
        pltpu.VMEM((CHUNK_E,), jnp.int32),
        pltpu.VMEM((CHUNK_E,), jnp.int32),
        pltpu.VMEM((CHUNK_E,), jnp.int32),
        pltpu.VMEM((CHUNK_E,), jnp.int32),
        pltpu.VMEM((TAIL_E,), jnp.int32),
        pltpu.VMEM((TAIL_E,), jnp.int32),
        pltpu.VMEM((CHUNK_E, H), jnp.float32),
        pltpu.VMEM((CHUNK_E, H), jnp.float32),
        pltpu.SemaphoreType.DMA,
        pltpu.SemaphoreType.DMA,
        pltpu.SemaphoreType.DMA,
        pltpu.SemaphoreType.DMA,
    ],
    compiler_params=_sc_params,
)
def _prop_kernel(hp_hbm, ei_hbm, zeros_hbm, out_hbm,
                 acc_sh, hp_sh, srcv0, srcv1, dstv0, dstv1, srcvt, dstvt,
                 rows0, rows1, gsem0, gsem1, ssem0, ssem1):
  cid = lax.axis_index("c")
  sid = lax.axis_index("s")
  tid = cid * NS + sid
  base = tid * MAIN_ROWS * 128
  srcv = (srcv0, srcv1)
  dstv = (dstv0, dstv1)
  rows = (rows0, rows1)
  gsem = (gsem0, gsem1)
  ssem = (ssem0, ssem1)
  m0 = jnp.where(tid == TAIL_OWNERS[0], 1.0, 0.0).astype(jnp.float32)
  m1 = jnp.where(tid == TAIL_OWNERS[1], 1.0, 0.0).astype(jnp.float32)

  pltpu.sync_copy(
      zeros_hbm, acc_sh.at[pl.ds(sid * ROWS_PER_SUB, ROWS_PER_SUB)])
  pltpu.sync_copy(
      hp_hbm.at[pl.ds(sid * ROWS_PER_SUB, ROWS_PER_SUB)],
      hp_sh.at[pl.ds(sid * ROWS_PER_SUB, ROWS_PER_SUB)])
  plsc.subcore_barrier()

  def load_idx(c, b):
    off = base + c * CHUNK_E
    pltpu.sync_copy(ei_hbm.at[0, pl.ds(off, CHUNK_E)], srcv[b])
    pltpu.sync_copy(ei_hbm.at[1, pl.ds(off, CHUNK_E)], dstv[b])

  load_idx(0, 0)
  gpend = [None, None]
  spend = [None, None]
  gpend[0] = pltpu.async_copy(hp_sh.at[srcv[0]], rows[0], gsem[0])
  for c in range(N_CHUNKS):
    b = c % 2
    gpend[b].wait()
    spend[b] = pltpu.async_copy(
        rows[b], acc_sh.at[dstv[b]], ssem[b], add=True)
    if c + 1 < N_CHUNKS:
      if spend[1 - b] is not None:
        spend[1 - b].wait()
      load_idx(c + 1, 1 - b)
      gpend[1 - b] = pltpu.async_copy(
          hp_sh.at[srcv[1 - b]], rows[1 - b], gsem[1 - b])
  spend[0].wait()
  spend[1].wait()

  pltpu.sync_copy(ei_hbm.at[0, pl.ds(TAIL_E0, TAIL_E)], srcvt)
  pltpu.sync_copy(ei_hbm.at[1, pl.ds(TAIL_E0, TAIL_E)], dstvt)
  pltpu.async_copy(
      hp_sh.at[srcvt], rows0.at[pl.ds(0, TAIL_E)], gsem[0]).wait()

  def mbody(i, carry):
    rows0[i, :] = rows0[i, :] * m0
    rows0[256 + i, :] = rows0[256 + i, :] * m1
    return carry
  lax.fori_loop(0, TAIL_E // 2, mbody, 0)
  pltpu.async_copy(
      rows0.at[pl.ds(0, TAIL_E)], acc_sh.at[dstvt], ssem[0],
      add=True).wait()

  plsc.subcore_barrier()
  pltpu.sync_copy(
      acc_sh.at[pl.ds(sid * ROWS_PER_SUB, ROWS_PER_SUB)],
      out_hbm.at[cid, pl.ds(sid * ROWS_PER_SUB, ROWS_PER_SUB)],
  )


NP8 = NPAD // 8


def _mm_body(x_ref, w_ref, u_ref):
  u_ref[...] = jnp.dot(x_ref[...], w_ref[...],
                       preferred_element_type=jnp.float32)


def _scale_body(u_ref, deg_ref, hp_ref):
  deg = deg_ref[0] + deg_ref[1] + 1.0
  dis = 1.0 / jnp.sqrt(deg)
  hp_ref[...] = dis * u_ref[...]


def _tc2_body(acc_ref, hp_ref, deg_ref, w_ref, b_ref, out_ref):
  deg = deg_ref[0] + deg_ref[1] + 1.0
  dis = 1.0 / jnp.sqrt(deg)
  t = jnp.tanh(dis * (acc_ref[0] + acc_ref[1] + hp_ref[...]) + b_ref[...])
  out_ref[...] = dis * jnp.dot(t, w_ref[...],
                               preferred_element_type=jnp.float32)


def _tc3_body(acc_ref, hp_ref, deg_ref, b_ref, out_ref):
  deg = deg_ref[0] + deg_ref[1] + 1.0
  dis = 1.0 / jnp.sqrt(deg)
  out_ref[...] = jnp.tanh(
      dis * (acc_ref[0] + acc_ref[1] + hp_ref[...]) + b_ref[...])


def kernel(x, edge_index, W1, b1, W2, b2):
  f32 = jnp.float32
  ei = edge_index.astype(jnp.int32)
  x_p = jnp.concatenate([x, jnp.zeros((NPAD - N, D), f32)], axis=0)
  zeros_hbm = jnp.zeros((ROWS_PER_SUB, H), f32)
  ones_hbm = jnp.ones((CHUNK_E, H), f32)
  x8 = x_p.reshape(NP8, 8 * D)
  W1b = jnp.zeros((8 * D, 128), f32)
  W2b = jnp.zeros((128, 128), f32)
  for j in range(8):
    W1b = lax.dynamic_update_slice(W1b, W1, (j * D, j * H))
    W2b = lax.dynamic_update_slice(W2b, W2, (j * H, j * H))
  b1t = jnp.tile(b1, 8).reshape(1, 128)
  b2t = jnp.tile(b2, 8).reshape(1, 128)

  deg = _deg_kernel(ei, zeros_hbm, ones_hbm)
  deg8 = deg.reshape(NC, NP8, 128)

  u1_8 = pl.pallas_call(
      _mm_body,
      out_shape=jax.ShapeDtypeStruct((NP8, 128), f32),
  )(x8, W1b)
  hp1_8 = pl.pallas_call(
      _scale_body,
      out_shape=jax.ShapeDtypeStruct((NP8, 128), f32),
  )(u1_8, deg8)

  acc1 = _prop_kernel(hp1_8.reshape(NPAD, H), ei, zeros_hbm)

  hp2_8 = pl.pallas_call(
      _tc2_body,
      out_shape=jax.ShapeDtypeStruct((NP8, 128), f32),
  )(acc1.reshape(NC, NP8, 128), hp1_8, deg8, W2b, b1t)

  acc2 = _prop_kernel(hp2_8.reshape(NPAD, H), ei, zeros_hbm)

  out8 = pl.pallas_call(
      _tc3_body,
      out_shape=jax.ShapeDtypeStruct((NP8, 128), f32),
  )(acc2.reshape(NC, NP8, 128), hp2_8, deg8, b2t)

  return out8.reshape(NPAD, H)[:N]
